# initial kernel scaffold (unmeasured)
import jax
import jax.numpy as jnp
from jax import lax
from jax.experimental import pallas as pl
from jax.experimental.pallas import tpu as pltpu

N_DEV = 4
ROWS = 2048
COLS = 1024
MAXSEG = 576


def _a2a_body(send_ref, cnt_ref, recv_ref, cnt_out_ref, send_sems, recv_sems,
              csend_sems, crecv_sems):
    me = lax.axis_index("i")

    barrier_sem = pltpu.get_barrier_semaphore()
    for d in range(1, N_DEV):
        peer = lax.rem(me + d, N_DEV)
        pl.semaphore_signal(
            barrier_sem, inc=1,
            device_id=(peer,), device_id_type=pl.DeviceIdType.MESH,
        )
    pl.semaphore_wait(barrier_sem, N_DEV - 1)

    recv_ref[0] = send_ref[0]
    cnt_out_ref[0] = cnt_ref[0]

    rdmas = []
    for d in range(1, N_DEV):
        peer = lax.rem(me + d, N_DEV)
        rd = pltpu.make_async_remote_copy(
            src_ref=send_ref.at[d],
            dst_ref=recv_ref.at[d],
            send_sem=send_sems.at[d],
            recv_sem=recv_sems.at[d],
            device_id=(peer,),
            device_id_type=pl.DeviceIdType.MESH,
        )
        rd.start()
        rc = pltpu.make_async_remote_copy(
            src_ref=cnt_ref.at[d],
            dst_ref=cnt_out_ref.at[d],
            send_sem=csend_sems.at[d],
            recv_sem=crecv_sems.at[d],
            device_id=(peer,),
            device_id_type=pl.DeviceIdType.MESH,
        )
        rc.start()
        rdmas.append((rd, rc))

    for rd, rc in rdmas:
        rd.wait()
        rc.wait()


def kernel(x, dest):
    me = lax.axis_index("i")

    d_off = (dest.astype(jnp.int32) - me) % N_DEV
    order = jnp.argsort(d_off, stable=True)
    d_sorted = d_off[order]
    seg_sizes = jnp.zeros((N_DEV,), jnp.int32).at[d_off].add(1)
    seg_start = jnp.concatenate(
        [jnp.zeros((1,), jnp.int32), jnp.cumsum(seg_sizes)[:-1]]
    )
    within = jnp.arange(ROWS, dtype=jnp.int32) - seg_start[d_sorted]
    slot = d_sorted * MAXSEG + within
    sendbuf = (
        jnp.zeros((N_DEV * MAXSEG, COLS), jnp.float32)
        .at[slot].set(x[order])
        .reshape(N_DEV, MAXSEG, COLS)
    )
    counts_send = jnp.zeros((N_DEV, 8, 128), jnp.int32).at[:, 0, 0].set(seg_sizes)

    recvbuf, counts_recv = pl.pallas_call(
        _a2a_body,
        out_shape=[
            jax.ShapeDtypeStruct((N_DEV, MAXSEG, COLS), jnp.float32),
            jax.ShapeDtypeStruct((N_DEV, 8, 128), jnp.int32),
        ],
        in_specs=[
            pl.BlockSpec(memory_space=pltpu.VMEM),
            pl.BlockSpec(memory_space=pltpu.VMEM),
        ],
        out_specs=[
            pl.BlockSpec(memory_space=pltpu.VMEM),
            pl.BlockSpec(memory_space=pltpu.VMEM),
        ],
        scratch_shapes=[
            pltpu.SemaphoreType.DMA((N_DEV,)),
            pltpu.SemaphoreType.DMA((N_DEV,)),
            pltpu.SemaphoreType.DMA((N_DEV,)),
            pltpu.SemaphoreType.DMA((N_DEV,)),
        ],
        compiler_params=pltpu.CompilerParams(collective_id=0),
    )(sendbuf, counts_send)

    cnt_k = counts_recv[:, 0, 0]
    k_of_s = (me - jnp.arange(N_DEV, dtype=jnp.int32)) % N_DEV
    sz = cnt_k[k_of_s]
    csum = jnp.cumsum(sz)
    t = jnp.arange(ROWS, dtype=jnp.int32)
    s_t = jnp.searchsorted(csum, t, side="right")
    within_t = t - (csum[s_t] - sz[s_t])
    gidx = k_of_s[s_t] * MAXSEG + within_t
    return recvbuf.reshape(N_DEV * MAXSEG, COLS)[gidx]


# baseline (device time: 221902 ns/iter reference)
import jax
import jax.numpy as jnp
from jax import lax
from jax.experimental import pallas as pl
from jax.experimental.pallas import tpu as pltpu

N_DEV = 4
ROWS = 2048
COLS = 1024
K_SEMS = 8


def _counts_body(cin_ref, cout_ref, send_sems, recv_sems):
    me = lax.axis_index("i")

    barrier_sem = pltpu.get_barrier_semaphore()
    for d in range(1, N_DEV):
        pl.semaphore_signal(
            barrier_sem, inc=1,
            device_id=(lax.rem(me + d, N_DEV),),
            device_id_type=pl.DeviceIdType.MESH,
        )
    pl.semaphore_wait(barrier_sem, N_DEV - 1)

    cout_ref[0] = cin_ref[...]
    rdmas = []
    for d in range(1, N_DEV):
        rd = pltpu.make_async_remote_copy(
            src_ref=cin_ref,
            dst_ref=cout_ref.at[d],
            send_sem=send_sems.at[d],
            recv_sem=recv_sems.at[d],
            device_id=(lax.rem(me + d, N_DEV),),
            device_id_type=pl.DeviceIdType.MESH,
        )
        rd.start()
        rdmas.append(rd)
    for rd in rdmas:
        rd.wait()


def _scatter_body(doff_ref, drow_ref, nin_ref, x_ref, out_ref,
                  copy_sems, recv_sems):
    me = lax.axis_index("i")

    barrier_sem = pltpu.get_barrier_semaphore()
    for d in range(1, N_DEV):
        pl.semaphore_signal(
            barrier_sem, inc=1,
            device_id=(lax.rem(me + d, N_DEV),),
            device_id_type=pl.DeviceIdType.MESH,
        )
    pl.semaphore_wait(barrier_sem, N_DEV - 1)

    def issue(i, carry):
        d = doff_ref[i]
        row = drow_ref[i]
        s = lax.rem(i, K_SEMS)

        @pl.when(i >= K_SEMS)
        def _():
            pltpu.make_async_copy(
                x_ref.at[pl.ds(0, 1)], out_ref.at[pl.ds(0, 1)],
                copy_sems.at[s],
            ).wait()

        @pl.when(d == 0)
        def _():
            pltpu.make_async_copy(
                x_ref.at[pl.ds(i, 1)],
                out_ref.at[pl.ds(row, 1)],
                copy_sems.at[s],
            ).start()

        @pl.when(d != 0)
        def _():
            pltpu.make_async_remote_copy(
                src_ref=x_ref.at[pl.ds(i, 1)],
                dst_ref=out_ref.at[pl.ds(row, 1)],
                send_sem=copy_sems.at[s],
                recv_sem=recv_sems.at[d],
                device_id=(lax.rem(me + d, N_DEV),),
                device_id_type=pl.DeviceIdType.MESH,
            ).start()

        return carry

    lax.fori_loop(0, ROWS, issue, 0)

    for s in range(K_SEMS):
        pltpu.make_async_copy(
            x_ref.at[pl.ds(0, 1)], out_ref.at[pl.ds(0, 1)],
            copy_sems.at[s],
        ).wait()

    for d in range(1, N_DEV):
        def wait_one(j, carry, d=d):
            pltpu.make_async_remote_copy(
                src_ref=x_ref.at[pl.ds(0, 1)],
                dst_ref=out_ref.at[pl.ds(0, 1)],
                send_sem=copy_sems.at[0],
                recv_sem=recv_sems.at[d],
                device_id=(me,),
                device_id_type=pl.DeviceIdType.MESH,
            ).wait_recv()
            return carry

        lax.fori_loop(0, nin_ref[d], wait_one, 0)


def kernel(x, dest):
    me = lax.axis_index("i")
    dest = dest.astype(jnp.int32)

    onehot = (dest[:, None] == jnp.arange(N_DEV, dtype=jnp.int32)[None, :])
    cnt_mine = onehot.sum(axis=0).astype(jnp.int32)
    cin = jnp.zeros((8, 128), jnp.int32).at[0, :N_DEV].set(cnt_mine)

    cout = pl.pallas_call(
        _counts_body,
        out_shape=jax.ShapeDtypeStruct((N_DEV, 8, 128), jnp.int32),
        in_specs=[pl.BlockSpec(memory_space=pltpu.VMEM)],
        out_specs=pl.BlockSpec(memory_space=pltpu.VMEM),
        scratch_shapes=[
            pltpu.SemaphoreType.DMA((N_DEV,)),
            pltpu.SemaphoreType.DMA((N_DEV,)),
        ],
        compiler_params=pltpu.CompilerParams(collective_id=1),
    )(cin)

    s4 = cout[:, 0, :N_DEV]
    perm = (me - jnp.arange(N_DEV, dtype=jnp.int32)) % N_DEV
    M = s4[perm]
    n_in = jnp.take(s4, me, axis=1)
    mask = (jnp.arange(N_DEV, dtype=jnp.int32)[:, None] < me)
    base = (M * mask).sum(axis=0).astype(jnp.int32)
    cs = jnp.cumsum(onehot.astype(jnp.int32), axis=0)
    within = jnp.take_along_axis(cs, dest[:, None], axis=1)[:, 0] - 1
    dst_row = base[dest] + within
    d_off = (dest - me) % N_DEV

    return pl.pallas_call(
        _scatter_body,
        out_shape=jax.ShapeDtypeStruct((ROWS, COLS), jnp.float32),
        in_specs=[
            pl.BlockSpec(memory_space=pltpu.SMEM),
            pl.BlockSpec(memory_space=pltpu.SMEM),
            pl.BlockSpec(memory_space=pltpu.SMEM),
            pl.BlockSpec(memory_space=pltpu.VMEM),
        ],
        out_specs=pl.BlockSpec(memory_space=pltpu.VMEM),
        scratch_shapes=[
            pltpu.SemaphoreType.DMA((K_SEMS,)),
            pltpu.SemaphoreType.DMA((N_DEV,)),
        ],
        compiler_params=pltpu.CompilerParams(collective_id=0),
    )(d_off, dst_row, n_in, x)


# device time: 109995 ns/iter; 2.0174x vs baseline; 2.0174x over previous
import jax
import jax.numpy as jnp
from jax import lax
from jax.experimental import pallas as pl
from jax.experimental.pallas import tpu as pltpu

N_DEV = 4
ROWS = 2048
COLS = 1024
UNROLL = 4


def _counts_body(cin_ref, cout_ref, send_sems, recv_sems):
    me = lax.axis_index("i")

    barrier_sem = pltpu.get_barrier_semaphore()
    for d in range(1, N_DEV):
        pl.semaphore_signal(
            barrier_sem, inc=1,
            device_id=(lax.rem(me + d, N_DEV),),
            device_id_type=pl.DeviceIdType.MESH,
        )
    pl.semaphore_wait(barrier_sem, N_DEV - 1)

    cout_ref[0] = cin_ref[...]
    rdmas = []
    for d in range(1, N_DEV):
        rd = pltpu.make_async_remote_copy(
            src_ref=cin_ref,
            dst_ref=cout_ref.at[d],
            send_sem=send_sems.at[d],
            recv_sem=recv_sems.at[d],
            device_id=(lax.rem(me + d, N_DEV),),
            device_id_type=pl.DeviceIdType.MESH,
        )
        rd.start()
        rdmas.append(rd)
    for rd in rdmas:
        rd.wait()


def _scatter_body(doff_ref, drow_ref, nin_ref, nout_ref, x_ref, out_ref,
                  send_sems, recv_sems):
    me = lax.axis_index("i")

    barrier_sem = pltpu.get_barrier_semaphore()
    for d in range(1, N_DEV):
        pl.semaphore_signal(
            barrier_sem, inc=1,
            device_id=(lax.rem(me + d, N_DEV),),
            device_id_type=pl.DeviceIdType.MESH,
        )
    pl.semaphore_wait(barrier_sem, N_DEV - 1)

    def issue(u, carry):
        for v in range(UNROLL):
            i = u * UNROLL + v
            d = doff_ref[i]
            row = drow_ref[i]
            pltpu.make_async_remote_copy(
                src_ref=x_ref.at[pl.ds(i, 1)],
                dst_ref=out_ref.at[pl.ds(row, 1)],
                send_sem=send_sems.at[d],
                recv_sem=recv_sems.at[d],
                device_id=(lax.rem(me + d, N_DEV),),
                device_id_type=pl.DeviceIdType.MESH,
            ).start()
        return carry

    lax.fori_loop(0, ROWS // UNROLL, issue, 0)

    for d in range(N_DEV):
        def wait_send_one(j, carry, d=d):
            pltpu.make_async_copy(
                x_ref.at[pl.ds(0, 1)], out_ref.at[pl.ds(0, 1)],
                send_sems.at[d],
            ).wait()
            return carry

        lax.fori_loop(0, nout_ref[d], wait_send_one, 0)

    for d in range(N_DEV):
        def wait_recv_one(j, carry, d=d):
            pltpu.make_async_remote_copy(
                src_ref=x_ref.at[pl.ds(0, 1)],
                dst_ref=out_ref.at[pl.ds(0, 1)],
                send_sem=send_sems.at[0],
                recv_sem=recv_sems.at[d],
                device_id=(me,),
                device_id_type=pl.DeviceIdType.MESH,
            ).wait_recv()
            return carry

        lax.fori_loop(0, nin_ref[d], wait_recv_one, 0)


def kernel(x, dest):
    me = lax.axis_index("i")
    dest = dest.astype(jnp.int32)
    j4 = jnp.arange(N_DEV, dtype=jnp.int32)

    onehot = (dest[:, None] == j4[None, :]).astype(jnp.int32)
    cnt_mine = onehot.sum(axis=0)
    cin = jnp.zeros((8, 128), jnp.int32).at[0, :N_DEV].set(cnt_mine)

    cout = pl.pallas_call(
        _counts_body,
        out_shape=jax.ShapeDtypeStruct((N_DEV, 8, 128), jnp.int32),
        in_specs=[pl.BlockSpec(memory_space=pltpu.VMEM)],
        out_specs=pl.BlockSpec(memory_space=pltpu.VMEM),
        scratch_shapes=[
            pltpu.SemaphoreType.DMA((N_DEV,)),
            pltpu.SemaphoreType.DMA((N_DEV,)),
        ],
        compiler_params=pltpu.CompilerParams(collective_id=1),
    )(cin)

    s4 = cout[:, 0, :N_DEV]
    M = s4[(me - j4) % N_DEV]
    n_in = (s4 * (j4[None, :] == me)).sum(axis=1)
    nout = cnt_mine[(me + j4) % N_DEV]
    base = (M * (j4[:, None] < me)).sum(axis=0)
    cs = jnp.cumsum(onehot, axis=0)
    within = (cs * onehot).sum(axis=1) - 1
    dst_row = (base[None, :] * onehot).sum(axis=1) + within
    d_off = (dest - me) % N_DEV

    return pl.pallas_call(
        _scatter_body,
        out_shape=jax.ShapeDtypeStruct((ROWS, COLS), jnp.float32),
        in_specs=[
            pl.BlockSpec(memory_space=pltpu.SMEM),
            pl.BlockSpec(memory_space=pltpu.SMEM),
            pl.BlockSpec(memory_space=pltpu.SMEM),
            pl.BlockSpec(memory_space=pltpu.SMEM),
            pl.BlockSpec(memory_space=pltpu.VMEM),
        ],
        out_specs=pl.BlockSpec(memory_space=pltpu.VMEM),
        scratch_shapes=[
            pltpu.SemaphoreType.DMA((N_DEV,)),
            pltpu.SemaphoreType.DMA((N_DEV,)),
        ],
        compiler_params=pltpu.CompilerParams(collective_id=0),
    )(d_off, dst_row, n_in, nout, x)
